# Initial kernel scaffold; baseline (speedup 1.0000x reference)
import jax
import jax.numpy as jnp
from jax.experimental import pallas as pl


def kernel(x, edge_index, W, b, gamma, beta):
    raise NotImplementedError("write your pallas kernel here")


# speedup vs baseline: 22.5071x; 22.5071x over previous
import jax
import jax.numpy as jnp
from jax import lax
from jax.experimental import pallas as pl
from jax.experimental.pallas import tpu as pltpu
from jax.experimental.pallas import tpu_sc as plsc

_N = 10000
_E = 320000
_D = 128
_NC = 2
_NS = 16
_NW = _NC * _NS
_N_PAD = 10240
_RPT = _N_PAD // _NS
_K = 128
_EPT = _E // _NW
_NCH = -(-_EPT // _K)
_PAD_E = _NCH * _K - _EPT


def _deg_body(packed, zeros1, ones1, pdeg, sdeg, idx2, ones_v):
    c = lax.axis_index("c")
    s = lax.axis_index("s")
    w = c * _NS + s
    pltpu.sync_copy(zeros1.at[pl.ds(s * _RPT, _RPT)], sdeg.at[pl.ds(s * _RPT, _RPT)])
    pltpu.sync_copy(ones1, ones_v)
    plsc.subcore_barrier()

    def chunk(k, carry):
        cid = w * _NCH + k
        pltpu.sync_copy(packed.at[cid], idx2)
        pltpu.sync_copy(ones_v, sdeg.at[idx2.at[1]], add=True)
        return carry

    lax.fori_loop(0, _NCH, chunk, 0)
    plsc.subcore_barrier()
    pltpu.sync_copy(sdeg.at[pl.ds(s * _RPT, _RPT)],
                    pdeg.at[pl.ds(c * _N_PAD + s * _RPT, _RPT)])


def _scat_body(g, packed, zeros2, pout, acc, idx2, rows):
    c = lax.axis_index("c")
    s = lax.axis_index("s")
    w = c * _NS + s
    pltpu.sync_copy(zeros2, acc.at[pl.ds(s * _RPT, _RPT)])
    plsc.subcore_barrier()

    def chunk(k, carry):
        cid = w * _NCH + k
        pltpu.sync_copy(packed.at[cid], idx2)
        pltpu.sync_copy(g.at[idx2.at[0]], rows)
        pltpu.sync_copy(rows, acc.at[idx2.at[1]], add=True)
        return carry

    lax.fori_loop(0, _NCH, chunk, 0)
    plsc.subcore_barrier()
    pltpu.sync_copy(acc.at[pl.ds(s * _RPT, _RPT)],
                    pout.at[pl.ds(c * _N_PAD + s * _RPT, _RPT)])


def _mm_body(x_ref, d2_ref, w_ref, g_ref):
    d2 = d2_ref[...]
    dinv = lax.rsqrt(d2[:, 0] + d2[:, 1] + 1.0)
    h = jnp.dot(x_ref[...], w_ref[...], preferred_element_type=jnp.float32)
    g_ref[...] = h * dinv[:, None]


def _fin_body(pout_ref, g_ref, d2_ref, b_ref, gam_ref, bet_ref, o_ref):
    pc = pout_ref[...]
    ssum = pc[:_N_PAD] + pc[_N_PAD:] + g_ref[...]
    d2 = d2_ref[...]
    dinv = lax.rsqrt(d2[:, 0] + d2[:, 1] + 1.0)
    pre = ssum * dinv[:, None] + b_ref[...]
    rid = lax.broadcasted_iota(jnp.int32, (_N_PAD, _D), 0)
    m = rid < _N
    mean = jnp.sum(jnp.where(m, pre, 0.0), axis=0) / _N
    dev = jnp.where(m, pre - mean[None, :], 0.0)
    var = jnp.sum(dev * dev, axis=0) / _N
    o = (pre - mean[None, :]) * lax.rsqrt(var + 1e-5) * gam_ref[...] + bet_ref[...]
    o_ref[...] = jnp.maximum(o, 0.0)


def _pack_edges(edge_index):
    i32 = jnp.int32
    padv = _N + (jnp.arange(_PAD_E, dtype=i32) % (_N_PAD - _N))
    padw = jnp.broadcast_to(padv, (_NW, _PAD_E))
    srcw = jnp.concatenate([edge_index[0].reshape(_NW, _EPT), padw], axis=1)
    dstw = jnp.concatenate([edge_index[1].reshape(_NW, _EPT), padw], axis=1)
    packed = jnp.stack(
        [srcw.reshape(_NW, _NCH, _K), dstw.reshape(_NW, _NCH, _K)], axis=2)
    return packed.reshape(_NW * _NCH, 2, _K)


def kernel(x, edge_index, W, b, gamma, beta):
    f32 = jnp.float32
    packed = _pack_edges(edge_index)
    zeros1 = jnp.zeros((_N_PAD,), f32)
    ones1 = jnp.ones((_K,), f32)
    zeros2 = jnp.zeros((_RPT, _D), f32)

    mesh = plsc.VectorSubcoreMesh(core_axis_name="c", subcore_axis_name="s")

    pdeg = pl.kernel(
        _deg_body,
        out_type=jax.ShapeDtypeStruct((2 * _N_PAD,), f32),
        mesh=mesh,
        scratch_types=[
            pltpu.VMEM_SHARED((_N_PAD,), f32),
            pltpu.VMEM((2, _K), jnp.int32),
            pltpu.VMEM((_K,), f32),
        ],
    )(packed, zeros1, ones1)
    d2 = pdeg.reshape(2, _N_PAD).T

    x_pad = jnp.pad(x, ((0, _N_PAD - _N), (0, 0)))
    bn = 512
    g = pl.pallas_call(
        _mm_body,
        grid=(_N_PAD // bn,),
        in_specs=[
            pl.BlockSpec((bn, _D), lambda i: (i, 0)),
            pl.BlockSpec((bn, 2), lambda i: (i, 0)),
            pl.BlockSpec((_D, _D), lambda i: (0, 0)),
        ],
        out_specs=pl.BlockSpec((bn, _D), lambda i: (i, 0)),
        out_shape=jax.ShapeDtypeStruct((_N_PAD, _D), f32),
    )(x_pad, d2, W)

    pout = pl.kernel(
        _scat_body,
        out_type=jax.ShapeDtypeStruct((2 * _N_PAD, _D), f32),
        mesh=mesh,
        scratch_types=[
            pltpu.VMEM_SHARED((_N_PAD, _D), f32),
            pltpu.VMEM((2, _K), jnp.int32),
            pltpu.VMEM((_K, _D), f32),
        ],
    )(g, packed, zeros2)

    out = pl.pallas_call(
        _fin_body,
        out_shape=jax.ShapeDtypeStruct((_N_PAD, _D), f32),
    )(pout, g, d2, b.reshape(1, _D), gamma.reshape(1, _D), beta.reshape(1, _D))
    return out[:_N]


# speedup vs baseline: 32.8246x; 1.4584x over previous
import jax
import jax.numpy as jnp
from jax import lax
from jax.experimental import pallas as pl
from jax.experimental.pallas import tpu as pltpu
from jax.experimental.pallas import tpu_sc as plsc

_N = 10000
_E = 320000
_D = 128
_NC = 2
_NS = 16
_NW = _NC * _NS
_N_PAD = 10240
_RPT = _N_PAD // _NS
_K = 112
_EPT = _E // _NW
_NCH = 90
_PAD_E = _NCH * _K - _EPT
_NBUF = 2
_ZB = 128


def _deg_body(packed, zeros1, ones1, pdeg, sdeg, idxa, ones_v, *dsem):
    c = lax.axis_index("c")
    s = lax.axis_index("s")
    w = c * _NS + s
    pltpu.sync_copy(zeros1.at[pl.ds(s * _RPT, _RPT)], sdeg.at[pl.ds(s * _RPT, _RPT)])
    pltpu.sync_copy(ones1, ones_v)
    pltpu.sync_copy(packed.at[pl.ds(w * 2 * _NCH + _NCH, _NCH)], idxa)
    plsc.subcore_barrier()

    def sdesc(k, b):
        return pltpu.make_async_copy(ones_v, sdeg.at[idxa.at[k, 0]], dsem[b])

    def rnd(r, carry):
        for b in range(_NBUF):
            k = _NBUF * r + b

            @pl.when(k >= _NBUF)
            def _():
                sdesc(k - _NBUF, b).wait()

            sdesc(k, b).start(add=True)
        return carry

    lax.fori_loop(0, _NCH // _NBUF, rnd, 0)
    for b in range(_NBUF):
        sdesc(_NCH - _NBUF + b, b).wait()
    plsc.subcore_barrier()
    pltpu.sync_copy(sdeg.at[pl.ds(s * _RPT, _RPT)],
                    pdeg.at[pl.ds(c * _N_PAD + s * _RPT, _RPT)])


def _scat_body(g, packed, zeros2, pout, acc, *sems):
    def scoped(idxa, rows):
        _scat_inner(g, packed, zeros2, pout, acc, idxa, rows, sems)

    pl.run_scoped(
        scoped,
        idxa=pltpu.VMEM((2 * _NCH, 1, _K), jnp.int32),
        rows=[pltpu.VMEM((_K, _D), jnp.float32) for _ in range(_NBUF)],
    )


def _scat_inner(g, packed, zeros2, pout, acc, idxa, rows, sems):
    gsem = sems[:_NBUF]
    ssem = sems[_NBUF:]
    c = lax.axis_index("c")
    s = lax.axis_index("s")
    w = c * _NS + s
    pltpu.sync_copy(packed.at[pl.ds(w * 2 * _NCH, 2 * _NCH)], idxa)
    for j in range(_RPT // _ZB):
        pltpu.sync_copy(zeros2, acc.at[pl.ds(s * _RPT + j * _ZB, _ZB)])
    plsc.subcore_barrier()

    def gdesc(k, b):
        return pltpu.make_async_copy(g.at[idxa.at[k, 0]], rows[b], gsem[b])

    def sdesc(k, b):
        return pltpu.make_async_copy(rows[b], acc.at[idxa.at[_NCH + k, 0]], ssem[b])

    gdesc(0, 0).start()
    gdesc(1, 1).start()

    def rnd(r, carry):
        for b in range(_NBUF):
            k = _NBUF * r + b
            bb = (b + 2) % _NBUF
            gdesc(k, b).wait()
            sdesc(k, b).start(add=True)

            @pl.when(k >= _NBUF - 2)
            def _():
                sdesc(k + 2 - _NBUF, bb).wait()

            @pl.when(k + 2 < _NCH)
            def _():
                gdesc(k + 2, bb).start()
        return carry

    lax.fori_loop(0, _NCH // _NBUF, rnd, 0)
    for j in range(_NCH + 2 - _NBUF, _NCH):
        sdesc(j, j % _NBUF).wait()
    plsc.subcore_barrier()
    for j in range(_RPT // _ZB):
        pltpu.sync_copy(acc.at[pl.ds(s * _RPT + j * _ZB, _ZB)],
                        pout.at[pl.ds(c * _N_PAD + s * _RPT + j * _ZB, _ZB)])


def _mm_body(x_ref, d2_ref, w_ref, g_ref):
    d2 = d2_ref[...]
    dinv = lax.rsqrt(d2[:, 0] + d2[:, 1] + 1.0)
    h = jnp.dot(x_ref[...], w_ref[...], preferred_element_type=jnp.float32)
    g_ref[...] = h * dinv[:, None]


def _fin_body(pout_ref, g_ref, d2_ref, b_ref, gam_ref, bet_ref, o_ref):
    pc = pout_ref[...]
    ssum = pc[:_N_PAD] + pc[_N_PAD:] + g_ref[...]
    d2 = d2_ref[...]
    dinv = lax.rsqrt(d2[:, 0] + d2[:, 1] + 1.0)
    pre = ssum * dinv[:, None] + b_ref[...]
    rid = lax.broadcasted_iota(jnp.int32, (_N_PAD, _D), 0)
    m = rid < _N
    mean = jnp.sum(jnp.where(m, pre, 0.0), axis=0) / _N
    dev = jnp.where(m, pre - mean[None, :], 0.0)
    var = jnp.sum(dev * dev, axis=0) / _N
    o = (pre - mean[None, :]) * lax.rsqrt(var + 1e-5) * gam_ref[...] + bet_ref[...]
    o_ref[...] = jnp.maximum(o, 0.0)


def _pack_edges(edge_index):
    i32 = jnp.int32
    padv = _N + (jnp.arange(_PAD_E, dtype=i32) % (_N_PAD - _N))
    padw = jnp.broadcast_to(padv, (_NW, _PAD_E))
    srcw = jnp.concatenate([edge_index[0].reshape(_NW, _EPT), padw], axis=1)
    dstw = jnp.concatenate([edge_index[1].reshape(_NW, _EPT), padw], axis=1)
    packed = jnp.stack(
        [srcw.reshape(_NW, _NCH, _K), dstw.reshape(_NW, _NCH, _K)], axis=1)
    return packed.reshape(_NW * _NCH * 2, 1, _K)


def kernel(x, edge_index, W, b, gamma, beta):
    f32 = jnp.float32
    packed = _pack_edges(edge_index)
    zeros1 = jnp.zeros((_N_PAD,), f32)
    ones1 = jnp.ones((_K,), f32)
    zeros2 = jnp.zeros((_ZB, _D), f32)

    mesh = plsc.VectorSubcoreMesh(core_axis_name="c", subcore_axis_name="s",
                                  num_cores=_NC, num_subcores=_NS)
    dma = pltpu.SemaphoreType.DMA

    pdeg = pl.kernel(
        _deg_body,
        out_type=jax.ShapeDtypeStruct((2 * _N_PAD,), f32),
        mesh=mesh,
        compiler_params=pltpu.CompilerParams(use_tc_tiling_on_sc=False),
        scratch_types=[
            pltpu.VMEM_SHARED((_N_PAD,), f32),
            pltpu.VMEM((_NCH, 1, _K), jnp.int32),
            pltpu.VMEM((_K,), f32),
        ] + [dma] * _NBUF,
    )(packed, zeros1, ones1)
    d2 = pdeg.reshape(2, _N_PAD).T

    x_pad = jnp.pad(x, ((0, _N_PAD - _N), (0, 0)))
    bn = 512
    g = pl.pallas_call(
        _mm_body,
        grid=(_N_PAD // bn,),
        in_specs=[
            pl.BlockSpec((bn, _D), lambda i: (i, 0)),
            pl.BlockSpec((bn, 2), lambda i: (i, 0)),
            pl.BlockSpec((_D, _D), lambda i: (0, 0)),
        ],
        out_specs=pl.BlockSpec((bn, _D), lambda i: (i, 0)),
        out_shape=jax.ShapeDtypeStruct((_N_PAD, _D), f32),
    )(x_pad, d2, W)

    pout = pl.kernel(
        _scat_body,
        out_type=jax.ShapeDtypeStruct((2 * _N_PAD, _D), f32),
        mesh=mesh,
        compiler_params=pltpu.CompilerParams(use_tc_tiling_on_sc=False),
        scratch_types=[
            pltpu.VMEM_SHARED((_N_PAD, _D), f32),
        ] + [dma] * (2 * _NBUF),
    )(g, packed, zeros2)

    out = pl.pallas_call(
        _fin_body,
        out_shape=jax.ShapeDtypeStruct((_N_PAD, _D), f32),
    )(pout, g, d2, b.reshape(1, _D), gamma.reshape(1, _D), beta.reshape(1, _D))
    return out[:_N]


# speedup vs baseline: 38.1593x; 1.1625x over previous
import jax
import jax.numpy as jnp
from jax import lax
from jax.experimental import pallas as pl
from jax.experimental.pallas import tpu as pltpu
from jax.experimental.pallas import tpu_sc as plsc

_N = 10000
_E = 320000
_D = 128
_NC = 2
_NS = 16
_NW = _NC * _NS
_N_PAD = 10240
_RPT = _N_PAD // _NS
_K = 72
_EPT = _E // _NW
_NCH = 141
_PAD_E = _NCH * _K - _EPT
_NBUF = 3
_ZB = 128


def _deg_body(packed, zeros1, ones1, pdeg, sdeg, idxa, ones_v, *dsem):
    c = lax.axis_index("c")
    s = lax.axis_index("s")
    w = c * _NS + s
    pltpu.sync_copy(zeros1.at[pl.ds(s * _RPT, _RPT)], sdeg.at[pl.ds(s * _RPT, _RPT)])
    pltpu.sync_copy(ones1, ones_v)
    pltpu.sync_copy(packed.at[pl.ds(w * 2 * _NCH + _NCH, _NCH)], idxa)
    plsc.subcore_barrier()

    def sdesc(k, b):
        return pltpu.make_async_copy(ones_v, sdeg.at[idxa.at[k, 0]], dsem[b])

    def rnd(r, carry):
        for b in range(_NBUF):
            k = _NBUF * r + b

            @pl.when(k >= _NBUF)
            def _():
                sdesc(k - _NBUF, b).wait()

            sdesc(k, b).start(add=True)
        return carry

    lax.fori_loop(0, _NCH // _NBUF, rnd, 0)
    for b in range(_NBUF):
        sdesc(_NCH - _NBUF + b, b).wait()
    plsc.subcore_barrier()
    pltpu.sync_copy(sdeg.at[pl.ds(s * _RPT, _RPT)],
                    pdeg.at[pl.ds(c * _N_PAD + s * _RPT, _RPT)])


def _scat_body(g, packed, zeros2, pout, acc, *sems):
    def scoped(idxa, rows):
        _scat_inner(g, packed, zeros2, pout, acc, idxa, rows, sems)

    pl.run_scoped(
        scoped,
        idxa=pltpu.VMEM((2 * _NCH, 1, _K), jnp.int32),
        rows=[pltpu.VMEM((_K, _D), jnp.float32) for _ in range(_NBUF)],
    )


def _scat_inner(g, packed, zeros2, pout, acc, idxa, rows, sems):
    gsem = sems[:_NBUF]
    ssem = sems[_NBUF:]
    c = lax.axis_index("c")
    s = lax.axis_index("s")
    w = c * _NS + s
    pltpu.sync_copy(packed.at[pl.ds(w * 2 * _NCH, 2 * _NCH)], idxa)
    for j in range(_RPT // _ZB):
        pltpu.sync_copy(zeros2, acc.at[pl.ds(s * _RPT + j * _ZB, _ZB)])
    plsc.subcore_barrier()

    def gdesc(k, b):
        return pltpu.make_async_copy(g.at[idxa.at[k, 0]], rows[b], gsem[b])

    def sdesc(k, b):
        return pltpu.make_async_copy(rows[b], acc.at[idxa.at[_NCH + k, 0]], ssem[b])

    gdesc(0, 0).start()
    gdesc(1, 1).start()

    def rnd(r, carry):
        for b in range(_NBUF):
            k = _NBUF * r + b
            bb = (b + 2) % _NBUF
            gdesc(k, b).wait()
            sdesc(k, b).start(add=True)

            @pl.when(k >= _NBUF - 2)
            def _():
                sdesc(k + 2 - _NBUF, bb).wait()

            @pl.when(k + 2 < _NCH)
            def _():
                gdesc(k + 2, bb).start()
        return carry

    lax.fori_loop(0, _NCH // _NBUF, rnd, 0)
    for j in range(_NCH + 2 - _NBUF, _NCH):
        sdesc(j, j % _NBUF).wait()
    plsc.subcore_barrier()
    for j in range(_RPT // _ZB):
        pltpu.sync_copy(acc.at[pl.ds(s * _RPT + j * _ZB, _ZB)],
                        pout.at[pl.ds(c * _N_PAD + s * _RPT + j * _ZB, _ZB)])


def _mm_body(x_ref, w_ref, h_ref):
    h_ref[...] = jnp.dot(x_ref[...], w_ref[...],
                         preferred_element_type=jnp.float32)


def _scale_body(h_ref, d2_ref, g_ref):
    d2 = d2_ref[...]
    dinv = lax.rsqrt(d2[:, 0] + d2[:, 1] + 1.0)
    g_ref[...] = h_ref[...] * dinv[:, None]


def _fin_body(pout_ref, g_ref, d2_ref, b_ref, gam_ref, bet_ref, o_ref):
    pc = pout_ref[...]
    ssum = pc[:_N_PAD] + pc[_N_PAD:] + g_ref[...]
    d2 = d2_ref[...]
    dinv = lax.rsqrt(d2[:, 0] + d2[:, 1] + 1.0)
    pre = ssum * dinv[:, None] + b_ref[...]
    rid = lax.broadcasted_iota(jnp.int32, (_N_PAD, _D), 0)
    m = rid < _N
    mean = jnp.sum(jnp.where(m, pre, 0.0), axis=0) / _N
    dev = jnp.where(m, pre - mean[None, :], 0.0)
    var = jnp.sum(dev * dev, axis=0) / _N
    o = (pre - mean[None, :]) * lax.rsqrt(var + 1e-5) * gam_ref[...] + bet_ref[...]
    o_ref[...] = jnp.maximum(o, 0.0)


def _pack_edges(edge_index):
    i32 = jnp.int32
    padv = _N + (jnp.arange(_PAD_E, dtype=i32) % (_N_PAD - _N))
    padw = jnp.broadcast_to(padv, (_NW, _PAD_E))
    srcw = jnp.concatenate([edge_index[0].reshape(_NW, _EPT), padw], axis=1)
    dstw = jnp.concatenate([edge_index[1].reshape(_NW, _EPT), padw], axis=1)
    packed = jnp.stack(
        [srcw.reshape(_NW, _NCH, _K), dstw.reshape(_NW, _NCH, _K)], axis=1)
    return packed.reshape(_NW * _NCH * 2, 1, _K)


def kernel(x, edge_index, W, b, gamma, beta):
    f32 = jnp.float32
    packed = _pack_edges(edge_index)
    zeros1 = jnp.zeros((_N_PAD,), f32)
    ones1 = jnp.ones((_K,), f32)
    zeros2 = jnp.zeros((_ZB, _D), f32)

    mesh = plsc.VectorSubcoreMesh(core_axis_name="c", subcore_axis_name="s",
                                  num_cores=_NC, num_subcores=_NS)
    dma = pltpu.SemaphoreType.DMA

    pdeg = pl.kernel(
        _deg_body,
        out_type=jax.ShapeDtypeStruct((2 * _N_PAD,), f32),
        mesh=mesh,
        compiler_params=pltpu.CompilerParams(use_tc_tiling_on_sc=False),
        scratch_types=[
            pltpu.VMEM_SHARED((_N_PAD,), f32),
            pltpu.VMEM((_NCH, 1, _K), jnp.int32),
            pltpu.VMEM((_K,), f32),
        ] + [dma] * _NBUF,
    )(packed, zeros1, ones1)
    d2 = pdeg.reshape(2, _N_PAD).T

    x_pad = jnp.pad(x, ((0, _N_PAD - _N), (0, 0)))
    bn = 2048
    h = pl.pallas_call(
        _mm_body,
        grid=(_N_PAD // bn,),
        in_specs=[
            pl.BlockSpec((bn, _D), lambda i: (i, 0)),
            pl.BlockSpec((_D, _D), lambda i: (0, 0)),
        ],
        out_specs=pl.BlockSpec((bn, _D), lambda i: (i, 0)),
        out_shape=jax.ShapeDtypeStruct((_N_PAD, _D), f32),
    )(x_pad, W)
    g = pl.pallas_call(
        _scale_body,
        grid=(_N_PAD // bn,),
        in_specs=[
            pl.BlockSpec((bn, _D), lambda i: (i, 0)),
            pl.BlockSpec((bn, 2), lambda i: (i, 0)),
        ],
        out_specs=pl.BlockSpec((bn, _D), lambda i: (i, 0)),
        out_shape=jax.ShapeDtypeStruct((_N_PAD, _D), f32),
    )(h, d2)

    pout = pl.kernel(
        _scat_body,
        out_type=jax.ShapeDtypeStruct((2 * _N_PAD, _D), f32),
        mesh=mesh,
        compiler_params=pltpu.CompilerParams(use_tc_tiling_on_sc=False),
        scratch_types=[
            pltpu.VMEM_SHARED((_N_PAD, _D), f32),
        ] + [dma] * (2 * _NBUF),
    )(g, packed, zeros2)

    out = pl.pallas_call(
        _fin_body,
        out_shape=jax.ShapeDtypeStruct((_N_PAD, _D), f32),
    )(pout, g, d2, b.reshape(1, _D), gamma.reshape(1, _D), beta.reshape(1, _D))
    return out[:_N]
